# baseline (speedup 1.0000x reference)
import functools

import jax
import jax.numpy as jnp
from jax import lax
from jax.experimental import pallas as pl
from jax.experimental.pallas import tpu as pltpu
from jax.experimental.pallas import tpu_sc as plsc

B = 16384
D = 64
BSC = B // 2
BTC = B - BSC
NUM_WORKERS = 32
BPW = BSC // NUM_WORKERS
CHUNK = 256
NCHUNK = BPW // CHUNK


def _body(uids_hbm, pids_hbm, utab_hbm, ptab_hbm, out_hbm,
          idx_u, idx_p, urows, prows, outv, sem_u, sem_p):
    wid = lax.axis_index("s") * 2 + lax.axis_index("c")
    base = wid * BPW

    pltpu.sync_copy(uids_hbm.at[pl.ds(base, BPW)], idx_u)
    pltpu.sync_copy(pids_hbm.at[pl.ds(base, BPW)], idx_p)

    lanes = lax.iota(jnp.int32, 16)

    def c_body(c, carry):
        def f_body(g, fc):
            iu = idx_u[pl.ds(c * CHUNK + g * 16, 16)]
            ip = idx_p[pl.ds(c * CHUNK + g * 16, 16)]
            for j in range(16):
                jj = g * 16 + j
                pltpu.async_copy(utab_hbm.at[iu[j]], urows.at[jj], sem_u)
                pltpu.async_copy(ptab_hbm.at[ip[j]], prows.at[jj], sem_p)
            return fc

        lax.fori_loop(0, CHUNK // 16, f_body, 0)

        pltpu.make_async_copy(utab_hbm.at[pl.ds(0, CHUNK)], urows,
                              sem_u).wait()
        pltpu.make_async_copy(ptab_hbm.at[pl.ds(0, CHUNK)], prows,
                              sem_p).wait()

        for g in range(CHUNK // 16):
            local = g * 16 + lanes

            def d_body(d, acc):
                cols = jnp.zeros((16,), jnp.int32) + d
                u = plsc.load_gather(urows, [local, cols])
                p = plsc.load_gather(prows, [local, cols])
                return acc + u * p

            acc = lax.fori_loop(0, D, d_body, jnp.zeros((16,), jnp.float32),
                                unroll=8)
            outv[pl.ds(c * CHUNK + g * 16, 16)] = acc
        return carry

    lax.fori_loop(0, NCHUNK, c_body, 0)
    pltpu.sync_copy(outv, out_hbm.at[pl.ds(base, BPW)])


def _tc_body(tu, su, tp, sp, u_ref, p_ref, o_ref):
    del tu, tp
    i = pl.program_id(0)
    u = u_ref[su[i], :]
    p = p_ref[sp[i], :]
    o_ref[pl.ds(i, 1), :] = jnp.sum(u * p).reshape(1, 1)


def _tc_gather_dot(uids, pids, user_table, place_table):
    grid_spec = pltpu.PrefetchScalarGridSpec(
        num_scalar_prefetch=4,
        grid=(BTC,),
        in_specs=[
            pl.BlockSpec((8, D), lambda i, tu, su, tp, sp: (tu[i], 0)),
            pl.BlockSpec((8, D), lambda i, tu, su, tp, sp: (tp[i], 0)),
        ],
        out_specs=pl.BlockSpec((BTC, 1), lambda i, tu, su, tp, sp: (0, 0)),
    )
    return pl.pallas_call(
        _tc_body,
        grid_spec=grid_spec,
        out_shape=jax.ShapeDtypeStruct((BTC, 1), jnp.float32),
    )(uids >> 3, uids & 7, pids >> 3, pids & 7, user_table, place_table)


@jax.jit
def _run(uids, pids, user_table, place_table):
    mesh = plsc.VectorSubcoreMesh(core_axis_name="c", subcore_axis_name="s")
    k = functools.partial(
        pl.kernel,
        mesh=mesh,
        compiler_params=pltpu.CompilerParams(
            needs_layout_passes=False, use_tc_tiling_on_sc=True),
        out_type=jax.ShapeDtypeStruct((BSC,), jnp.float32),
        scratch_types=[
            pltpu.VMEM((BPW,), jnp.int32),
            pltpu.VMEM((BPW,), jnp.int32),
            pltpu.VMEM((CHUNK, D), jnp.float32),
            pltpu.VMEM((CHUNK, D), jnp.float32),
            pltpu.VMEM((BPW,), jnp.float32),
            pltpu.SemaphoreType.DMA,
            pltpu.SemaphoreType.DMA,
        ],
    )(_body)
    out_sc = k(uids[:BSC], pids[:BSC], user_table, place_table)
    out_tc = _tc_gather_dot(uids[BSC:], pids[BSC:], user_table, place_table)
    return jnp.concatenate([out_sc.reshape(BSC, 1), out_tc], axis=0)


def kernel(inputs, user_table, place_table):
    uids = inputs[:, 0].astype(jnp.int32)
    pids = inputs[:, 1].astype(jnp.int32)
    return _run(uids, pids, user_table, place_table)


# speedup vs baseline: 5.8435x; 5.8435x over previous
import functools

import jax
import jax.numpy as jnp
from jax import lax
from jax.experimental import pallas as pl
from jax.experimental.pallas import tpu as pltpu
from jax.experimental.pallas import tpu_sc as plsc

B = 16384
D = 64
NUM_WORKERS = 32
BPW = B // NUM_WORKERS
CHUNK = 256
NCHUNK = BPW // CHUNK


def _body(uids_hbm, pids_hbm, utab_hbm, ptab_hbm, out_hbm,
          idx_u, idx_p, urows, prows, outv, sem_u, sem_p):
    wid = lax.axis_index("s") * 2 + lax.axis_index("c")
    base = wid * BPW

    pltpu.sync_copy(uids_hbm.at[pl.ds(base, BPW)], idx_u)
    pltpu.sync_copy(pids_hbm.at[pl.ds(base, BPW)], idx_p)

    lanes = lax.iota(jnp.int32, 16)

    def c_body(c, carry):
        def f_body(g, fc):
            iu = idx_u[pl.ds(c * CHUNK + g * 16, 16)]
            ip = idx_p[pl.ds(c * CHUNK + g * 16, 16)]
            for j in range(16):
                jj = g * 16 + j
                pltpu.async_copy(utab_hbm.at[iu[j]], urows.at[jj], sem_u)
                pltpu.async_copy(ptab_hbm.at[ip[j]], prows.at[jj], sem_p)
            return fc

        lax.fori_loop(0, CHUNK // 16, f_body, 0)

        pltpu.make_async_copy(utab_hbm.at[pl.ds(0, CHUNK)], urows,
                              sem_u).wait()
        pltpu.make_async_copy(ptab_hbm.at[pl.ds(0, CHUNK)], prows,
                              sem_p).wait()

        for g in range(CHUNK // 16):
            local = g * 16 + lanes

            def d_body(d, acc):
                cols = jnp.zeros((16,), jnp.int32) + d
                u = plsc.load_gather(urows, [local, cols])
                p = plsc.load_gather(prows, [local, cols])
                return acc + u * p

            acc = lax.fori_loop(0, D, d_body, jnp.zeros((16,), jnp.float32),
                                unroll=8)
            outv[pl.ds(c * CHUNK + g * 16, 16)] = acc
        return carry

    lax.fori_loop(0, NCHUNK, c_body, 0)
    pltpu.sync_copy(outv, out_hbm.at[pl.ds(base, BPW)])


@jax.jit
def _run(uids, pids, user_table, place_table):
    mesh = plsc.VectorSubcoreMesh(core_axis_name="c", subcore_axis_name="s")
    k = functools.partial(
        pl.kernel,
        mesh=mesh,
        compiler_params=pltpu.CompilerParams(
            needs_layout_passes=False, use_tc_tiling_on_sc=True,
            disable_bounds_checks=True, disable_semaphore_checks=True),
        out_type=jax.ShapeDtypeStruct((B,), jnp.float32),
        scratch_types=[
            pltpu.VMEM((BPW,), jnp.int32),
            pltpu.VMEM((BPW,), jnp.int32),
            pltpu.VMEM((CHUNK, D), jnp.float32),
            pltpu.VMEM((CHUNK, D), jnp.float32),
            pltpu.VMEM((BPW,), jnp.float32),
            pltpu.SemaphoreType.DMA,
            pltpu.SemaphoreType.DMA,
        ],
    )(_body)
    return k(uids, pids, user_table, place_table)


def kernel(inputs, user_table, place_table):
    uids = inputs[:, 0].astype(jnp.int32)
    pids = inputs[:, 1].astype(jnp.int32)
    out = _run(uids, pids, user_table, place_table)
    return out.reshape(B, 1)
